# baseline (speedup 1.0000x reference)
import functools

import jax
import jax.numpy as jnp
from jax import lax
from jax.experimental import pallas as pl
from jax.experimental.pallas import tpu as pltpu
from jax.experimental.pallas import tpu_sc as plsc


def kernel(x, pos_emb):
    seq_len = x.shape[1]
    dim = pos_emb.shape[1]
    info = plsc.get_sparse_core_info()
    num_workers = info.num_cores * info.num_subcores
    rows_per_w = seq_len // num_workers

    mesh = plsc.VectorSubcoreMesh(core_axis_name="c", subcore_axis_name="s")

    @functools.partial(
        pl.kernel,
        mesh=mesh,
        out_type=jax.ShapeDtypeStruct((seq_len, dim), pos_emb.dtype),
    )
    def copy_k(pos_hbm, out_hbm):
        wid = lax.axis_index("s") * info.num_cores + lax.axis_index("c")
        base = wid * rows_per_w
        pltpu.sync_copy(
            pos_hbm.at[pl.ds(base, rows_per_w), :],
            out_hbm.at[pl.ds(base, rows_per_w), :],
        )

    return copy_k(pos_emb)


# speedup vs baseline: 25.0708x; 25.0708x over previous
import functools

import jax
import jax.numpy as jnp
from jax import lax
from jax.experimental import pallas as pl
from jax.experimental.pallas import tpu as pltpu
from jax.experimental.pallas import tpu_sc as plsc

_CHUNK = 16
_NBUF = 3


def kernel(x, pos_emb):
    seq_len = x.shape[1]
    dim = pos_emb.shape[1]
    info = plsc.get_sparse_core_info()
    num_workers = info.num_cores * info.num_subcores
    rows_per_w = seq_len // num_workers
    n_chunks = rows_per_w // _CHUNK

    mesh = plsc.VectorSubcoreMesh(core_axis_name="c", subcore_axis_name="s")

    @functools.partial(
        pl.kernel,
        mesh=mesh,
        out_type=jax.ShapeDtypeStruct((seq_len, dim), pos_emb.dtype),
        scratch_types=[
            pltpu.VMEM((_NBUF, _CHUNK, dim), pos_emb.dtype),
            pltpu.SemaphoreType.DMA((_NBUF,)),
            pltpu.SemaphoreType.DMA((_NBUF,)),
        ],
    )
    def copy_k(pos_hbm, out_hbm, buf, insem, outsem):
        wid = lax.axis_index("s") * info.num_cores + lax.axis_index("c")
        base = wid * rows_per_w
        ins = []
        outs = []
        for i in range(n_chunks):
            b = i % _NBUF
            ins.append(
                pltpu.make_async_copy(
                    pos_hbm.at[pl.ds(base + i * _CHUNK, _CHUNK), :],
                    buf.at[b],
                    insem.at[b],
                )
            )
            outs.append(
                pltpu.make_async_copy(
                    buf.at[b],
                    out_hbm.at[pl.ds(base + i * _CHUNK, _CHUNK), :],
                    outsem.at[b],
                )
            )
        for i in range(min(_NBUF, n_chunks)):
            ins[i].start()
        for i in range(n_chunks):
            ins[i].wait()
            outs[i].start()
            j = i + _NBUF
            if j < n_chunks:
                outs[i].wait()
                ins[j].start()
        for i in range(max(0, n_chunks - _NBUF), n_chunks):
            outs[i].wait()

    return copy_k(pos_emb)


# speedup vs baseline: 49.4338x; 1.9718x over previous
import jax
import jax.numpy as jnp
from jax.experimental import pallas as pl
from jax.experimental.pallas import tpu as pltpu

_N_CHUNKS = 8


def _staged_copy(src_ref, out_ref, bufs, insems, outsems):
    rows = out_ref.shape[0]
    chunk = rows // _N_CHUNKS
    ins = [
        pltpu.make_async_copy(
            src_ref.at[pl.ds(i * chunk, chunk), :], bufs.at[i], insems.at[i]
        )
        for i in range(_N_CHUNKS)
    ]
    outs = [
        pltpu.make_async_copy(
            bufs.at[i], out_ref.at[pl.ds(i * chunk, chunk), :], outsems.at[i]
        )
        for i in range(_N_CHUNKS)
    ]
    for c in ins:
        c.start()
    for i in range(_N_CHUNKS):
        ins[i].wait()
        outs[i].start()
    for c in outs:
        c.wait()


def kernel(x, pos_emb):
    seq_len = x.shape[1]
    dim = pos_emb.shape[1]
    chunk = seq_len // _N_CHUNKS
    return pl.pallas_call(
        _staged_copy,
        in_specs=[pl.BlockSpec(memory_space=pl.ANY)],
        out_specs=pl.BlockSpec(memory_space=pl.ANY),
        scratch_shapes=[
            pltpu.VMEM((_N_CHUNKS, chunk, dim), pos_emb.dtype),
            pltpu.SemaphoreType.DMA((_N_CHUNKS,)),
            pltpu.SemaphoreType.DMA((_N_CHUNKS,)),
        ],
        out_shape=jax.ShapeDtypeStruct((seq_len, dim), pos_emb.dtype),
    )(pos_emb)
